# Initial kernel scaffold; baseline (speedup 1.0000x reference)
import jax
import jax.numpy as jnp
from jax.experimental import pallas as pl


def kernel(constraint_features, edge_indices, edge_features, variable_features, bbounds, params):
    raise NotImplementedError("write your pallas kernel here")


# speedup vs baseline: 29.9815x; 29.9815x over previous
import functools

import jax
import jax.numpy as jnp
from jax import lax
from jax.experimental import pallas as pl
from jax.experimental.pallas import tpu as pltpu
from jax.experimental.pallas import tpu_sc as plsc

NV = 100000
NC = 100000
NE = 1600000
EMB = 32
NP = 102400
G = 16
NRG = NP // G
BMG = 128
NGG = NRG // BMG

NT = 16
CH = 80
KB = 10
E_TILE = NE // NT
R_TILE = E_TILE // CH
NB = E_TILE // (CH * KB)


def _ln(x, w, b):
    m = jnp.mean(x, axis=-1, keepdims=True)
    v = jnp.mean((x - m) ** 2, axis=-1, keepdims=True)
    return (x - m) / jnp.sqrt(v + 1e-5) * w + b


def _dot(a, b):
    return jax.lax.dot_general(a, b, (((1,), (0,)), ((), ())),
                               preferred_element_type=jnp.float32,
                               precision=jax.lax.Precision.HIGHEST)


def _gelu(x):
    return 0.5 * x * (1.0 + lax.erf(x * 0.7071067811865476))


def _kron16(w):
    return jnp.kron(jnp.eye(16, dtype=jnp.float32), w)


def _tile16(v):
    return jnp.tile(v.reshape(-1), 16).reshape(1, -1)


def _sc_round(y2, idx2, ew, zeros, d):
    n = y2.shape[1]
    ntile = n // NT
    log2d = d.bit_length() - 1
    rpg = 16 // d
    ng = CH * d // 16
    mesh = plsc.VectorSubcoreMesh(core_axis_name="c", subcore_axis_name="s")

    @functools.partial(
        pl.kernel,
        out_type=jax.ShapeDtypeStruct((2, NT, ntile, d), jnp.float32),
        mesh=mesh,
        compiler_params=pltpu.CompilerParams(use_tc_tiling_on_sc=False,
                                             needs_layout_passes=False),
        scratch_types=[
            pltpu.VMEM((2, KB, CH), jnp.int32),
            pltpu.VMEM((2, KB, CH), jnp.int32),
            pltpu.VMEM((2, KB * CH), jnp.float32),
            pltpu.VMEM((2, KB, CH, d), jnp.float32),
            pltpu.SemaphoreType.DMA,
            pltpu.SemaphoreType.DMA,
            pltpu.SemaphoreType.DMA,
            pltpu.SemaphoreType.DMA,
            pltpu.SemaphoreType.DMA,
            pltpu.VMEM_SHARED((n, d), jnp.float32),
        ],
    )
    def k(y2_hbm, idx2_hbm, ew_hbm, z_hbm, out_hbm,
          gidx, sidx, ewb, rows, gsem, ssem, sgsem, sssem, sesem, acc):
        c = lax.axis_index("c")
        t = lax.axis_index("s")
        pltpu.sync_copy(z_hbm.at[t],
                        acc.at[pl.ds(t * ntile, ntile), :])
        plsc.subcore_barrier()

        iot = lax.iota(jnp.int32, 16)
        pat_r = lax.shift_right_logical(iot, log2d)
        pat_c = lax.bitwise_and(iot, d - 1)

        def issue_gidx(b, s):
            rb = t * R_TILE + b * KB
            pltpu.async_copy(idx2_hbm.at[c, pl.ds(rb, KB), :],
                             gidx.at[s], sgsem)

        def issue_sidx(b, s):
            rb = t * R_TILE + b * KB
            pltpu.async_copy(idx2_hbm.at[1 - c, pl.ds(rb, KB), :],
                             sidx.at[s], sssem)

        def issue_ewb(b, s):
            eb = t * E_TILE + b * (KB * CH)
            pltpu.async_copy(ew_hbm.at[pl.ds(eb, KB * CH)],
                             ewb.at[s], sesem)

        def wait_gidx(s):
            pltpu.make_async_copy(idx2_hbm.at[0, pl.ds(0, KB), :],
                                  gidx.at[s], sgsem).wait()

        def wait_sidx(s):
            pltpu.make_async_copy(idx2_hbm.at[0, pl.ds(0, KB), :],
                                  sidx.at[s], sssem).wait()

        def wait_ewb(s):
            pltpu.make_async_copy(ew_hbm.at[pl.ds(0, KB * CH)],
                                  ewb.at[s], sesem).wait()

        def fire_gathers(s):
            for kk in range(KB):
                pltpu.async_copy(y2_hbm.at[c].at[gidx.at[s, kk]],
                                 rows.at[s, kk], gsem)

        def wait_gathers(s):
            for kk in range(KB):
                pltpu.make_async_copy(y2_hbm.at[c].at[gidx.at[s, kk]],
                                      rows.at[s, kk], gsem).wait()

        def fire_scatters(s):
            for kk in range(KB):
                pltpu.async_copy(rows.at[s, kk], acc.at[sidx.at[s, kk]],
                                 ssem, add=True)

        def wait_scatters(s):
            for kk in range(KB):
                pltpu.make_async_copy(rows.at[s, kk], acc.at[sidx.at[s, kk]],
                                      ssem).wait()

        def compute(s):
            sv = jnp.full((16,), s, jnp.int32)
            for kk in range(KB):
                kv = jnp.full((16,), kk, jnp.int32)
                ebase = kk * CH

                @plsc.parallel_loop(0, ng, 1, unroll=4)
                def _(g, sv=sv, kv=kv, ebase=ebase):
                    ir = pat_r + g * rpg
                    w16 = plsc.load_gather(ewb, [sv, ir + ebase])
                    v16 = plsc.load_gather(rows, [sv, kv, ir, pat_c])
                    plsc.store_scatter(rows, [sv, kv, ir, pat_c], v16 * w16)

        issue_gidx(0, 0)
        issue_ewb(0, 0)
        issue_sidx(0, 0)
        wait_gidx(0)
        wait_ewb(0)
        fire_gathers(0)
        issue_gidx(1, 1)
        issue_ewb(1, 1)

        def batch(b, carry):
            slot = lax.rem(b, 2)
            other = 1 - slot
            wait_gathers(slot)

            @pl.when(b >= 1)
            def _():
                wait_scatters(other)

            @pl.when(b + 1 < NB)
            def _():
                wait_gidx(other)
                wait_ewb(other)
                fire_gathers(other)

            compute(slot)
            wait_sidx(slot)
            fire_scatters(slot)

            @pl.when(b + 1 < NB)
            def _():
                issue_sidx(b + 1, other)

            @pl.when(b + 2 < NB)
            def _():
                issue_gidx(b + 2, slot)
                issue_ewb(b + 2, slot)

            return carry

        lax.fori_loop(0, NB, batch, 0)
        wait_scatters(lax.rem(NB - 1, 2))
        plsc.subcore_barrier()
        pltpu.sync_copy(acc.at[pl.ds(t * ntile, ntile), :],
                        out_hbm.at[c, t])

    return k(y2, idx2, ew, zeros)


def _full(shape):
    return pl.BlockSpec(shape, lambda i: (0,) * len(shape))


def _tc_ew(ef2, w_ee, b_ee):
    rows, cols = ef2.shape
    bm = 8

    def body(ef_ref, w_ref, b_ref, o_ref):
        o_ref[...] = ef_ref[...] * w_ref[0, 0] + b_ref[0, 0]

    return pl.pallas_call(
        body,
        grid=(rows // bm,),
        in_specs=[pl.BlockSpec((bm, cols), lambda i: (i, 0)),
                  _full((1, 1)), _full((1, 1))],
        out_specs=pl.BlockSpec((bm, cols), lambda i: (i, 0)),
        out_shape=jax.ShapeDtypeStruct((rows, cols), jnp.float32),
    )(ef2, w_ee, b_ee.reshape(1, 1))


def _row_spec(cols):
    return pl.BlockSpec((BMG, cols), lambda i: (i, 0))


def _tc_embed(vf_g, cf_g, p):
    mv = _kron16(jnp.full((10, 10), 0.1, jnp.float32))
    mc = _kron16(jnp.full((6, 6), 1.0 / 6.0, jnp.float32))
    wve = _kron16(p["W_ve"])
    wce = _kron16(p["W_ce"])
    wr1v = _kron16(p["v2c"][0][0])
    wr1c = _kron16(p["c2v"][0][0])

    def emb(x, m_ref, lnw, lnb, w_ref, b_ref):
        x0 = x[...]
        mu = _dot(x0, m_ref[...])
        xc = x0 - mu
        var = _dot(xc * xc, m_ref[...])
        xn = xc * lax.rsqrt(var + 1e-5) * lnw[...] + lnb[...]
        return jax.nn.relu(_dot(xn, w_ref[...]) + b_ref[...])

    def body(vf_ref, cf_ref, mv_ref, lnvw, lnvb, wve_ref, bve,
             mc_ref, lncw, lncb, wce_ref, bce, wrv_ref, wrc_ref,
             var_ref, cons_ref, y_ref):
        v = emb(vf_ref, mv_ref, lnvw, lnvb, wve_ref, bve)
        c = emb(cf_ref, mc_ref, lncw, lncb, wce_ref, bce)
        var_ref[...] = v
        cons_ref[...] = c
        y_ref[0] = _dot(v, wrv_ref[...])
        y_ref[1] = _dot(c, wrc_ref[...])

    return pl.pallas_call(
        body,
        grid=(NGG,),
        in_specs=[_row_spec(160), _row_spec(96),
                  _full((160, 160)), _full((1, 160)), _full((1, 160)),
                  _full((160, 512)), _full((1, 512)),
                  _full((96, 96)), _full((1, 96)), _full((1, 96)),
                  _full((96, 512)), _full((1, 512)),
                  _full((512, 128)), _full((512, 128))],
        out_specs=[_row_spec(512), _row_spec(512),
                   pl.BlockSpec((2, BMG, 128), lambda i: (0, i, 0))],
        out_shape=[jax.ShapeDtypeStruct((NRG, 512), jnp.float32),
                   jax.ShapeDtypeStruct((NRG, 512), jnp.float32),
                   jax.ShapeDtypeStruct((2, NRG, 128), jnp.float32)],
    )(vf_g, cf_g,
      mv, _tile16(p["ln_v_w"]), _tile16(p["ln_v_b"]),
      wve, _tile16(p["b_ve"]),
      mc, _tile16(p["ln_c_w"]), _tile16(p["ln_c_b"]),
      wce, _tile16(p["b_ce"]),
      wr1v, wr1c)


def _pad8(w):
    return jnp.pad(w, ((0, 8 - w.shape[0]), (0, 8 - w.shape[1])))


def _mid_mats(p, layer):
    _, br_v, wo_v = p["v2c"][layer]
    _, br_c, wo_c = p["c2v"][layer]
    if layer == 0:
        wov_k, woc_k = _kron16(wo_v), _kron16(wo_c)
    else:
        wov_k, woc_k = _kron16(_pad8(wo_v)), _kron16(_pad8(wo_c))
    brv_t = _tile16(jnp.pad(br_v, (0, 8 - br_v.shape[0])))
    brc_t = _tile16(jnp.pad(br_c, (0, 8 - br_c.shape[0])))
    return wov_k, brv_t, woc_k, brc_t


def _tc_mid(agg_g, varp_g, consp_g, p, layer):
    din_cols = varp_g.shape[1]
    wov_k, brv_t, woc_k, brc_t = _mid_mats(p, layer)
    wnv_k = _kron16(_pad8(p["v2c"][layer + 1][0]))
    wnc_k = _kron16(_pad8(p["c2v"][layer + 1][0]))

    def body(agg_ref, varp_ref, consp_ref, wov, brv, woc, brc, wnv, wnc,
             var_ref, cons_ref, y_ref):
        cn = _gelu(agg_ref[0] + brv[...] + _dot(consp_ref[...], wov[...]))
        vn = _gelu(agg_ref[1] + brc[...] + _dot(varp_ref[...], woc[...]))
        var_ref[...] = vn
        cons_ref[...] = cn
        y_ref[0] = _dot(vn, wnv[...])
        y_ref[1] = _dot(cn, wnc[...])

    return pl.pallas_call(
        body,
        grid=(NGG,),
        in_specs=[pl.BlockSpec((2, BMG, 128), lambda i: (0, i, 0)),
                  _row_spec(din_cols), _row_spec(din_cols),
                  _full((din_cols, 128)), _full((1, 128)),
                  _full((din_cols, 128)), _full((1, 128)),
                  _full((128, 128)), _full((128, 128))],
        out_specs=[_row_spec(128), _row_spec(128),
                   pl.BlockSpec((2, BMG, 128), lambda i: (0, i, 0))],
        out_shape=[jax.ShapeDtypeStruct((NRG, 128), jnp.float32),
                   jax.ShapeDtypeStruct((NRG, 128), jnp.float32),
                   jax.ShapeDtypeStruct((2, NRG, 128), jnp.float32)],
    )(agg_g, varp_g, consp_g, wov_k, brv_t, woc_k, brc_t, wnv_k, wnc_k)


def _tc_final(agg_g, varp_g, consp_g, bbounds, p):
    din_cols = varp_g.shape[1]
    wov_k, brv_t, woc_k, brc_t = _mid_mats(p, 2)
    fold = jnp.tile(jnp.eye(8, dtype=jnp.float32), (16, 1))

    def body(agg_ref, varp_ref, consp_ref, wov, brv, woc, brc, fold_ref,
             bb, lnbw, lnbb, wbe, bbe, wfp, bfp, lnfw, lnfb,
             o_ref, acc):
        i = pl.program_id(0)
        cn = _gelu(agg_ref[0] + brv[...] + _dot(consp_ref[...], wov[...]))
        vn = _gelu(agg_ref[1] + brc[...] + _dot(varp_ref[...], woc[...]))
        part = jnp.concatenate(
            [jnp.sum(vn, axis=0, keepdims=True),
             jnp.sum(cn, axis=0, keepdims=True)], axis=1)

        @pl.when(i == 0)
        def _():
            acc[...] = part

        @pl.when(i > 0)
        def _():
            acc[...] = acc[...] + part

        @pl.when(i == NGG - 1)
        def _():
            s = acc[...]
            sv = _dot(s[:, 0:128], fold_ref[...])
            sc = _dot(s[:, 128:256], fold_ref[...])
            va = sv[:, 0:4] * (1.0 / NV)
            ca = sc[:, 0:4] * (1.0 / NC)
            bounds = jax.nn.relu(
                _dot(_ln(bb[...], lnbw[...], lnbb[...]), wbe[...]) + bbe[...])
            g = jnp.concatenate([va, ca, bounds], axis=1)
            o_ref[...] = jax.nn.relu(
                _ln(_dot(g, wfp[...]) + bfp[...], lnfw[...], lnfb[...]))

    return pl.pallas_call(
        body,
        grid=(NGG,),
        in_specs=[pl.BlockSpec((2, BMG, 128), lambda i: (0, i, 0)),
                  _row_spec(din_cols), _row_spec(din_cols),
                  _full((din_cols, 128)), _full((1, 128)),
                  _full((din_cols, 128)), _full((1, 128)),
                  _full((128, 8)),
                  _full((1, 2)), _full((1, 2)), _full((1, 2)),
                  _full((2, 2)), _full((1, 2)),
                  _full((10, 15)), _full((1, 15)),
                  _full((1, 15)), _full((1, 15))],
        out_specs=pl.BlockSpec((1, 15), lambda i: (0, 0)),
        out_shape=jax.ShapeDtypeStruct((1, 15), jnp.float32),
        scratch_shapes=[pltpu.VMEM((1, 256), jnp.float32)],
    )(agg_g, varp_g, consp_g, wov_k, brv_t, woc_k, brc_t, fold,
      bbounds, p["ln_b_w"].reshape(1, 2), p["ln_b_b"].reshape(1, 2),
      p["W_be"], p["b_be"].reshape(1, 2),
      p["W_fp"], p["b_fp"].reshape(1, 15),
      p["ln_fp_w"].reshape(1, 15), p["ln_fp_b"].reshape(1, 15))


def kernel(constraint_features, edge_indices, edge_features,
           variable_features, bbounds, params):
    p = params
    idx2 = edge_indices.reshape(2, NE // CH, CH)
    ef2 = edge_features.reshape(128, NE // 128)
    ew = _tc_ew(ef2, p["W_ee"], p["b_ee"]).reshape(NE)

    vf_g = jnp.pad(variable_features, ((0, NP - NV), (0, 0))).reshape(NRG, 160)
    cf_g = jnp.pad(constraint_features, ((0, NP - NC), (0, 0))).reshape(NRG, 96)

    var_g, cons_g, y_g = _tc_embed(vf_g, cf_g, p)
    z8 = jnp.zeros((NT, NP // NT, 8), jnp.float32)

    agg = _sc_round(y_g.reshape(2, NP, 8), idx2, ew, z8, 8)
    var_g, cons_g, y_g = _tc_mid(agg.reshape(2, NRG, 128), var_g, cons_g, p, 0)
    agg = _sc_round(y_g.reshape(2, NP, 8), idx2, ew, z8, 8)
    var_g, cons_g, y_g = _tc_mid(agg.reshape(2, NRG, 128), var_g, cons_g, p, 1)
    agg = _sc_round(y_g.reshape(2, NP, 8), idx2, ew, z8, 8)
    return _tc_final(agg.reshape(2, NRG, 128), var_g, cons_g, bbounds, p)
